# Initial kernel scaffold; baseline (speedup 1.0000x reference)
import jax
import jax.numpy as jnp
from jax.experimental import pallas as pl


def kernel(x, edge_index, W1, b1, W2, b2, W3, b3):
    raise NotImplementedError("write your pallas kernel here")


# speedup vs baseline: 12.2177x; 12.2177x over previous
import functools

import jax
import jax.numpy as jnp
from jax import lax
from jax.experimental import pallas as pl
from jax.experimental.pallas import tpu as pltpu
from jax.experimental.pallas import tpu_sc as plsc

NC = 2
NS = 16
NW = NC * NS
LANES = 16
CHUNK = 80
ZR = 128
NP = 10240


def _seg_sum_sc(h, src, dst, n, d, e):
    ew = e // NW
    nchunk = ew // CHUNK
    rows_per_tile = NP // NS
    mesh = plsc.VectorSubcoreMesh(
        core_axis_name="c", subcore_axis_name="s", num_cores=NC, num_subcores=NS
    )

    @functools.partial(
        pl.kernel,
        out_type=jax.ShapeDtypeStruct((NC, NP, d), jnp.float32),
        mesh=mesh,
        scratch_types=[
            pltpu.VMEM((CHUNK,), jnp.int32),
            pltpu.VMEM((CHUNK,), jnp.int32),
            pltpu.VMEM((CHUNK, d), jnp.float32),
            pltpu.VMEM((ZR, d), jnp.float32),
            pltpu.VMEM_SHARED((NP, d), jnp.float32),
            pltpu.SemaphoreType.DMA,
        ],
        compiler_params=pltpu.CompilerParams(use_tc_tiling_on_sc=False),
    )
    def k(h_hbm, src_hbm, dst_hbm, out_hbm, sidx, didx, rows, zbuf, acc, sem):
        cid = lax.axis_index("c")
        sid = lax.axis_index("s")
        wid = sid * NC + cid
        z16 = jnp.zeros((LANES,), jnp.float32)

        def zrow(i, carry):
            for j in range(d // LANES):
                zbuf[i, pl.ds(j * LANES, LANES)] = z16
            return carry

        lax.fori_loop(0, ZR, zrow, 0)
        row0 = sid * rows_per_tile
        for kk in range(rows_per_tile // ZR):
            pltpu.sync_copy(zbuf, acc.at[pl.ds(row0 + kk * ZR, ZR)])
        plsc.subcore_barrier()

        eb = wid * ew

        def body(j, carry):
            base = eb + j * CHUNK
            pltpu.sync_copy(src_hbm.at[pl.ds(base, CHUNK)], sidx)
            pltpu.sync_copy(dst_hbm.at[pl.ds(base, CHUNK)], didx)
            pltpu.async_copy(h_hbm.at[sidx], rows, sem).wait()
            pltpu.sync_copy(rows, acc.at[didx], add=True)
            return carry

        lax.fori_loop(0, nchunk, body, 0)
        plsc.subcore_barrier()

        out_c = out_hbm.at[cid]
        for kk in range(rows_per_tile // ZR):
            r = row0 + kk * ZR
            pltpu.sync_copy(acc.at[pl.ds(r, ZR)], out_c.at[pl.ds(r, ZR)])

    return k(h, src, dst)


def _indeg_sc(dst, n, e):
    d = LANES
    ew = e // NW
    nchunk = ew // CHUNK
    rows_per_tile = NP // NS
    mesh = plsc.VectorSubcoreMesh(
        core_axis_name="c", subcore_axis_name="s", num_cores=NC, num_subcores=NS
    )

    @functools.partial(
        pl.kernel,
        out_type=jax.ShapeDtypeStruct((NC, NP, d), jnp.float32),
        mesh=mesh,
        scratch_types=[
            pltpu.VMEM((CHUNK,), jnp.int32),
            pltpu.VMEM((CHUNK, d), jnp.float32),
            pltpu.VMEM((ZR, d), jnp.float32),
            pltpu.VMEM_SHARED((NP, d), jnp.float32),
        ],
        compiler_params=pltpu.CompilerParams(use_tc_tiling_on_sc=False),
    )
    def k(dst_hbm, out_hbm, didx, ones, zbuf, acc):
        cid = lax.axis_index("c")
        sid = lax.axis_index("s")
        wid = sid * NC + cid
        z16 = jnp.zeros((LANES,), jnp.float32)
        o16 = jnp.ones((LANES,), jnp.float32)

        def zrow(i, carry):
            zbuf[i, pl.ds(0, LANES)] = z16
            return carry

        lax.fori_loop(0, ZR, zrow, 0)

        def orow(i, carry):
            ones[i, pl.ds(0, LANES)] = o16
            return carry

        lax.fori_loop(0, CHUNK, orow, 0)

        row0 = sid * rows_per_tile
        for kk in range(rows_per_tile // ZR):
            pltpu.sync_copy(zbuf, acc.at[pl.ds(row0 + kk * ZR, ZR)])
        plsc.subcore_barrier()

        eb = wid * ew

        def body(j, carry):
            base = eb + j * CHUNK
            pltpu.sync_copy(dst_hbm.at[pl.ds(base, CHUNK)], didx)
            pltpu.sync_copy(ones, acc.at[didx], add=True)
            return carry

        lax.fori_loop(0, nchunk, body, 0)
        plsc.subcore_barrier()

        out_c = out_hbm.at[cid]
        for kk in range(rows_per_tile // ZR):
            r = row0 + kk * ZR
            pltpu.sync_copy(acc.at[pl.ds(r, ZR)], out_c.at[pl.ds(r, ZR)])

    return k(dst)


_BN = 1000


def _tc_first(x, w, ind):
    n, din = x.shape
    dh = w.shape[1]

    def body(x_ref, w_ref, ind_ref, dis_ref, hs_ref):
        indeg = ind_ref[0, :, :1] + ind_ref[1, :, :1]
        dis = lax.rsqrt(indeg + 1.0)
        dis_ref[...] = dis
        h = jnp.dot(x_ref[...], w_ref[...], preferred_element_type=jnp.float32)
        hs_ref[...] = h * dis

    return pl.pallas_call(
        body,
        grid=(n // _BN,),
        in_specs=[
            pl.BlockSpec((_BN, din), lambda i: (i, 0)),
            pl.BlockSpec((din, dh), lambda i: (0, 0)),
            pl.BlockSpec((NC, _BN, LANES), lambda i: (0, i, 0)),
        ],
        out_specs=[
            pl.BlockSpec((_BN, 1), lambda i: (i, 0)),
            pl.BlockSpec((_BN, dh), lambda i: (i, 0)),
        ],
        out_shape=[
            jax.ShapeDtypeStruct((n, 1), jnp.float32),
            jax.ShapeDtypeStruct((n, dh), jnp.float32),
        ],
    )(x, w, ind)


def _tc_mid(s, hs, b, dis, w):
    n, dp = hs.shape
    dn = w.shape[1]

    def body(s_ref, hs_ref, b_ref, dis_ref, w_ref, out_ref):
        agg = s_ref[0] + s_ref[1] + hs_ref[...]
        xv = agg * dis_ref[...] + b_ref[...]
        xv = jnp.maximum(xv, 0.0)
        out_ref[...] = (
            jnp.dot(xv, w_ref[...], preferred_element_type=jnp.float32)
            * dis_ref[...]
        )

    return pl.pallas_call(
        body,
        grid=(n // _BN,),
        in_specs=[
            pl.BlockSpec((NC, _BN, dp), lambda i: (0, i, 0)),
            pl.BlockSpec((_BN, dp), lambda i: (i, 0)),
            pl.BlockSpec((1, dp), lambda i: (0, 0)),
            pl.BlockSpec((_BN, 1), lambda i: (i, 0)),
            pl.BlockSpec((dp, dn), lambda i: (0, 0)),
        ],
        out_specs=pl.BlockSpec((_BN, dn), lambda i: (i, 0)),
        out_shape=jax.ShapeDtypeStruct((n, dn), jnp.float32),
    )(s, hs, b, dis, w)


def _tc_final(s, hs, b, dis):
    n, dp = hs.shape

    def body(s_ref, hs_ref, b_ref, dis_ref, out_ref):
        agg = s_ref[0] + s_ref[1] + hs_ref[...]
        out_ref[...] = agg * dis_ref[...] + b_ref[...]

    return pl.pallas_call(
        body,
        grid=(n // _BN,),
        in_specs=[
            pl.BlockSpec((NC, _BN, dp), lambda i: (0, i, 0)),
            pl.BlockSpec((_BN, dp), lambda i: (i, 0)),
            pl.BlockSpec((1, dp), lambda i: (0, 0)),
            pl.BlockSpec((_BN, 1), lambda i: (i, 0)),
        ],
        out_specs=pl.BlockSpec((_BN, dp), lambda i: (i, 0)),
        out_shape=jax.ShapeDtypeStruct((n, dp), jnp.float32),
    )(s, hs, b, dis)


def kernel(x, edge_index, W1, b1, W2, b2, W3, b3):
    n, _ = x.shape
    e = edge_index.shape[1]
    src = edge_index[0]
    dst = edge_index[1]

    ind = _indeg_sc(dst, n, e)
    dis, hs1 = _tc_first(x, W1, ind)

    s1 = _seg_sum_sc(hs1, src, dst, n, hs1.shape[1], e)
    hs2 = _tc_mid(s1, hs1, b1.reshape(1, -1), dis, W2)

    s2 = _seg_sum_sc(hs2, src, dst, n, hs2.shape[1], e)
    hs3 = _tc_mid(s2, hs2, b2.reshape(1, -1), dis, W3)

    s3 = _seg_sum_sc(hs3, src, dst, n, hs3.shape[1], e)
    return _tc_final(s3, hs3, b3.reshape(1, -1), dis)


# speedup vs baseline: 13.0755x; 1.0702x over previous
import functools

import jax
import jax.numpy as jnp
from jax import lax
from jax.experimental import pallas as pl
from jax.experimental.pallas import tpu as pltpu
from jax.experimental.pallas import tpu_sc as plsc

NC = 2
NS = 16
NW = NC * NS
LANES = 16
CHUNK = 128
ZR = 128
NP = 10240
EP = 327680


def _seg_sum_sc(h, src2d, dst2d, d):
    ew = EP // NW
    nchunk = ew // CHUNK
    rows_per_tile = NP // NS
    mesh = plsc.VectorSubcoreMesh(
        core_axis_name="c", subcore_axis_name="s", num_cores=NC, num_subcores=NS
    )

    @functools.partial(
        pl.kernel,
        out_type=jax.ShapeDtypeStruct((NC, NP, d), jnp.float32),
        mesh=mesh,
        scratch_types=[
            pltpu.VMEM((nchunk, CHUNK), jnp.int32),
            pltpu.VMEM((nchunk, CHUNK), jnp.int32),
            pltpu.VMEM((CHUNK, d), jnp.float32),
            pltpu.VMEM((CHUNK, d), jnp.float32),
            pltpu.VMEM_SHARED((NP, d), jnp.float32),
            pltpu.SemaphoreType.DMA,
        ],
        compiler_params=pltpu.CompilerParams(use_tc_tiling_on_sc=False),
    )
    def k(h_hbm, src_hbm, dst_hbm, out_hbm, sidx, didx, rows0, rows1, acc, sem):
        cid = lax.axis_index("c")
        sid = lax.axis_index("s")
        wid = sid * NC + cid
        z16 = jnp.zeros((LANES,), jnp.float32)

        def zrow(i, carry):
            for j in range(d // LANES):
                rows0[i, pl.ds(j * LANES, LANES)] = z16
            return carry

        lax.fori_loop(0, ZR, zrow, 0)
        row0 = sid * rows_per_tile
        for kk in range(rows_per_tile // ZR):
            pltpu.sync_copy(rows0, acc.at[pl.ds(row0 + kk * ZR, ZR)])

        pltpu.sync_copy(src_hbm.at[pl.ds(wid * nchunk, nchunk)], sidx)
        pltpu.sync_copy(dst_hbm.at[pl.ds(wid * nchunk, nchunk)], didx)
        pltpu.async_copy(h_hbm.at[sidx.at[0]], rows0, sem)
        pltpu.async_copy(h_hbm.at[sidx.at[1]], rows1, sem)
        plsc.subcore_barrier()

        def outer(g, carry):
            for b, rbuf in ((0, rows0), (1, rows1)):
                jj = g * 2 + b
                pltpu.make_async_copy(h_hbm.at[sidx.at[jj]], rbuf, sem).wait()
                pltpu.sync_copy(rbuf, acc.at[didx.at[jj]], add=True)

                @pl.when(jj + 2 < nchunk)
                def _():
                    pltpu.async_copy(h_hbm.at[sidx.at[jj + 2]], rbuf, sem)

            return carry

        lax.fori_loop(0, nchunk // 2, outer, 0)
        plsc.subcore_barrier()

        out_c = out_hbm.at[cid]
        for kk in range(rows_per_tile // ZR):
            r = row0 + kk * ZR
            pltpu.sync_copy(acc.at[pl.ds(r, ZR)], out_c.at[pl.ds(r, ZR)])

    return k(h, src2d, dst2d)


def _seg_sum_sc_colsplit(h2, src2d, dst2d, d):
    ew = EP // NS
    nchunk = ew // CHUNK
    rows_per_tile = NP // NS
    mesh = plsc.VectorSubcoreMesh(
        core_axis_name="c", subcore_axis_name="s", num_cores=NC, num_subcores=NS
    )

    @functools.partial(
        pl.kernel,
        out_type=jax.ShapeDtypeStruct((NC, NP, d), jnp.float32),
        mesh=mesh,
        scratch_types=[
            pltpu.VMEM((nchunk, CHUNK), jnp.int32),
            pltpu.VMEM((nchunk, CHUNK), jnp.int32),
            pltpu.VMEM((CHUNK, d), jnp.float32),
            pltpu.VMEM((CHUNK, d), jnp.float32),
            pltpu.VMEM_SHARED((NP, d), jnp.float32),
            pltpu.SemaphoreType.DMA,
        ],
        compiler_params=pltpu.CompilerParams(use_tc_tiling_on_sc=False),
    )
    def k(h_hbm, src_hbm, dst_hbm, out_hbm, sidx, didx, rows0, rows1, acc, sem):
        cid = lax.axis_index("c")
        sid = lax.axis_index("s")
        h_c = h_hbm.at[cid]
        z16 = jnp.zeros((LANES,), jnp.float32)

        def zrow(i, carry):
            for j in range(d // LANES):
                rows0[i, pl.ds(j * LANES, LANES)] = z16
            return carry

        lax.fori_loop(0, ZR, zrow, 0)
        row0 = sid * rows_per_tile
        for kk in range(rows_per_tile // ZR):
            pltpu.sync_copy(rows0, acc.at[pl.ds(row0 + kk * ZR, ZR)])

        pltpu.sync_copy(src_hbm.at[pl.ds(sid * nchunk, nchunk)], sidx)
        pltpu.sync_copy(dst_hbm.at[pl.ds(sid * nchunk, nchunk)], didx)
        pltpu.async_copy(h_c.at[sidx.at[0]], rows0, sem)
        pltpu.async_copy(h_c.at[sidx.at[1]], rows1, sem)
        plsc.subcore_barrier()

        def outer(g, carry):
            for b, rbuf in ((0, rows0), (1, rows1)):
                jj = g * 2 + b
                pltpu.make_async_copy(h_c.at[sidx.at[jj]], rbuf, sem).wait()
                pltpu.sync_copy(rbuf, acc.at[didx.at[jj]], add=True)

                @pl.when(jj + 2 < nchunk)
                def _():
                    pltpu.async_copy(h_c.at[sidx.at[jj + 2]], rbuf, sem)

            return carry

        lax.fori_loop(0, nchunk // 2, outer, 0)
        plsc.subcore_barrier()

        out_c = out_hbm.at[cid]
        for kk in range(rows_per_tile // ZR):
            r = row0 + kk * ZR
            pltpu.sync_copy(acc.at[pl.ds(r, ZR)], out_c.at[pl.ds(r, ZR)])

    return k(h2, src2d, dst2d)


def _indeg_sc(dst2d):
    d = LANES
    ew = EP // NW
    nchunk = ew // CHUNK
    rows_per_tile = NP // NS
    mesh = plsc.VectorSubcoreMesh(
        core_axis_name="c", subcore_axis_name="s", num_cores=NC, num_subcores=NS
    )

    @functools.partial(
        pl.kernel,
        out_type=jax.ShapeDtypeStruct((NC, NP, d), jnp.float32),
        mesh=mesh,
        scratch_types=[
            pltpu.VMEM((nchunk, CHUNK), jnp.int32),
            pltpu.VMEM((CHUNK, d), jnp.float32),
            pltpu.VMEM((ZR, d), jnp.float32),
            pltpu.VMEM_SHARED((NP, d), jnp.float32),
        ],
        compiler_params=pltpu.CompilerParams(use_tc_tiling_on_sc=False),
    )
    def k(dst_hbm, out_hbm, didx, ones, zbuf, acc):
        cid = lax.axis_index("c")
        sid = lax.axis_index("s")
        wid = sid * NC + cid
        z16 = jnp.zeros((LANES,), jnp.float32)
        o16 = jnp.ones((LANES,), jnp.float32)

        def zrow(i, carry):
            zbuf[i, pl.ds(0, LANES)] = z16
            ones[i, pl.ds(0, LANES)] = o16
            return carry

        lax.fori_loop(0, ZR, zrow, 0)

        row0 = sid * rows_per_tile
        for kk in range(rows_per_tile // ZR):
            pltpu.sync_copy(zbuf, acc.at[pl.ds(row0 + kk * ZR, ZR)])
        pltpu.sync_copy(dst_hbm.at[pl.ds(wid * nchunk, nchunk)], didx)
        plsc.subcore_barrier()

        def body(j, carry):
            pltpu.sync_copy(ones, acc.at[didx.at[j]], add=True)
            return carry

        lax.fori_loop(0, nchunk, body, 0)
        plsc.subcore_barrier()

        out_c = out_hbm.at[cid]
        for kk in range(rows_per_tile // ZR):
            r = row0 + kk * ZR
            pltpu.sync_copy(acc.at[pl.ds(r, ZR)], out_c.at[pl.ds(r, ZR)])

    return k(dst2d)


_BN = 1000


def _tc_first(x, w, ind):
    n, din = x.shape
    dh = w.shape[1]
    hh = dh // 2

    def body(x_ref, w_ref, ind_ref, dis_ref, hs_ref):
        indeg = ind_ref[0, :, :1] + ind_ref[1, :, :1]
        dis = lax.rsqrt(indeg + 1.0)
        dis_ref[...] = dis
        h = jnp.dot(x_ref[...], w_ref[...], preferred_element_type=jnp.float32)
        hs = h * dis
        hs_ref[0] = hs[:, :hh]
        hs_ref[1] = hs[:, hh:]

    return pl.pallas_call(
        body,
        grid=(n // _BN,),
        in_specs=[
            pl.BlockSpec((_BN, din), lambda i: (i, 0)),
            pl.BlockSpec((din, dh), lambda i: (0, 0)),
            pl.BlockSpec((NC, _BN, LANES), lambda i: (0, i, 0)),
        ],
        out_specs=[
            pl.BlockSpec((_BN, 1), lambda i: (i, 0)),
            pl.BlockSpec((2, _BN, hh), lambda i: (0, i, 0)),
        ],
        out_shape=[
            jax.ShapeDtypeStruct((n, 1), jnp.float32),
            jax.ShapeDtypeStruct((2, n, hh), jnp.float32),
        ],
    )(x, w, ind)


def _tc_mid_split(s, hs2, b, dis, w):
    n = hs2.shape[1]
    dh = hs2.shape[2]
    dn = w.shape[1]

    def body(s_ref, hs_ref, b_ref, dis_ref, w_ref, out_ref):
        agg = jnp.concatenate(
            [s_ref[0] + hs_ref[0], s_ref[1] + hs_ref[1]], axis=-1
        )
        xv = agg * dis_ref[...] + b_ref[...]
        xv = jnp.maximum(xv, 0.0)
        out_ref[...] = (
            jnp.dot(xv, w_ref[...], preferred_element_type=jnp.float32)
            * dis_ref[...]
        )

    return pl.pallas_call(
        body,
        grid=(n // _BN,),
        in_specs=[
            pl.BlockSpec((NC, _BN, dh), lambda i: (0, i, 0)),
            pl.BlockSpec((2, _BN, dh), lambda i: (0, i, 0)),
            pl.BlockSpec((1, 2 * dh), lambda i: (0, 0)),
            pl.BlockSpec((_BN, 1), lambda i: (i, 0)),
            pl.BlockSpec((2 * dh, dn), lambda i: (0, 0)),
        ],
        out_specs=pl.BlockSpec((_BN, dn), lambda i: (i, 0)),
        out_shape=jax.ShapeDtypeStruct((n, dn), jnp.float32),
    )(s, hs2, b, dis, w)


def _tc_mid(s, hs, b, dis, w):
    n, dp = hs.shape
    dn = w.shape[1]

    def body(s_ref, hs_ref, b_ref, dis_ref, w_ref, out_ref):
        agg = s_ref[0] + s_ref[1] + hs_ref[...]
        xv = agg * dis_ref[...] + b_ref[...]
        xv = jnp.maximum(xv, 0.0)
        out_ref[...] = (
            jnp.dot(xv, w_ref[...], preferred_element_type=jnp.float32)
            * dis_ref[...]
        )

    return pl.pallas_call(
        body,
        grid=(n // _BN,),
        in_specs=[
            pl.BlockSpec((NC, _BN, dp), lambda i: (0, i, 0)),
            pl.BlockSpec((_BN, dp), lambda i: (i, 0)),
            pl.BlockSpec((1, dp), lambda i: (0, 0)),
            pl.BlockSpec((_BN, 1), lambda i: (i, 0)),
            pl.BlockSpec((dp, dn), lambda i: (0, 0)),
        ],
        out_specs=pl.BlockSpec((_BN, dn), lambda i: (i, 0)),
        out_shape=jax.ShapeDtypeStruct((n, dn), jnp.float32),
    )(s, hs, b, dis, w)


def _tc_final(s, hs, b, dis):
    n, dp = hs.shape

    def body(s_ref, hs_ref, b_ref, dis_ref, out_ref):
        agg = s_ref[0] + s_ref[1] + hs_ref[...]
        out_ref[...] = agg * dis_ref[...] + b_ref[...]

    return pl.pallas_call(
        body,
        grid=(n // _BN,),
        in_specs=[
            pl.BlockSpec((NC, _BN, dp), lambda i: (0, i, 0)),
            pl.BlockSpec((_BN, dp), lambda i: (i, 0)),
            pl.BlockSpec((1, dp), lambda i: (0, 0)),
            pl.BlockSpec((_BN, 1), lambda i: (i, 0)),
        ],
        out_specs=pl.BlockSpec((_BN, dp), lambda i: (i, 0)),
        out_shape=jax.ShapeDtypeStruct((n, dp), jnp.float32),
    )(s, hs, b, dis)


def kernel(x, edge_index, W1, b1, W2, b2, W3, b3):
    e = edge_index.shape[1]
    pad = EP - e
    src2d = jnp.concatenate(
        [edge_index[0], jnp.zeros((pad,), jnp.int32)]
    ).reshape(EP // CHUNK, CHUNK)
    dst2d = jnp.concatenate(
        [edge_index[1], jnp.full((pad,), NP - 1, jnp.int32)]
    ).reshape(EP // CHUNK, CHUNK)

    ind = _indeg_sc(dst2d)
    dis, hs1 = _tc_first(x, W1, ind)

    s1 = _seg_sum_sc_colsplit(hs1, src2d, dst2d, hs1.shape[2])
    hs2 = _tc_mid_split(s1, hs1, b1.reshape(1, -1), dis, W2)

    s2 = _seg_sum_sc(hs2, src2d, dst2d, hs2.shape[1])
    hs3 = _tc_mid(s2, hs2, b2.reshape(1, -1), dis, W3)

    s3 = _seg_sum_sc(hs3, src2d, dst2d, hs3.shape[1])
    return _tc_final(s3, hs3, b3.reshape(1, -1), dis)


# speedup vs baseline: 15.7035x; 1.2010x over previous
import functools

import jax
import jax.numpy as jnp
from jax import lax
from jax.experimental import pallas as pl
from jax.experimental.pallas import tpu as pltpu
from jax.experimental.pallas import tpu_sc as plsc

NC = 2
NS = 16
LANES = 16
DG = 32
CHUNK = 128
ZR = 128
NP = 10240
EP = 327680
NRING = 8


def _seg_sum_sc(h2, src2d, dst2d):
    d = DG
    ew = EP // NS
    nchunk = ew // CHUNK
    rows_per_tile = NP // NS
    mesh = plsc.VectorSubcoreMesh(
        core_axis_name="c", subcore_axis_name="s", num_cores=NC, num_subcores=NS
    )

    @functools.partial(
        pl.kernel,
        out_type=jax.ShapeDtypeStruct((NC, NP, d), jnp.float32),
        mesh=mesh,
        scratch_types=[
            pltpu.VMEM((nchunk, CHUNK), jnp.int32),
            pltpu.VMEM((nchunk, CHUNK), jnp.int32),
        ]
        + [pltpu.VMEM((CHUNK, d), jnp.float32)] * NRING
        + [
            pltpu.VMEM_SHARED((NP, d), jnp.float32),
            pltpu.SemaphoreType.DMA,
        ],
        compiler_params=pltpu.CompilerParams(use_tc_tiling_on_sc=False),
    )
    def k(h_hbm, src_hbm, dst_hbm, out_hbm, sidx, didx, *rest):
        rows = rest[:NRING]
        acc = rest[NRING]
        sem = rest[NRING + 1]
        cid = lax.axis_index("c")
        sid = lax.axis_index("s")
        h_c = h_hbm.at[cid]
        z16 = jnp.zeros((LANES,), jnp.float32)

        def zrow(i, carry):
            for j in range(d // LANES):
                rows[0][i, pl.ds(j * LANES, LANES)] = z16
            return carry

        lax.fori_loop(0, ZR, zrow, 0)
        row0 = sid * rows_per_tile
        for kk in range(rows_per_tile // ZR):
            pltpu.sync_copy(rows[0], acc.at[pl.ds(row0 + kk * ZR, ZR)])

        pltpu.sync_copy(src_hbm.at[pl.ds(sid * nchunk, nchunk)], sidx)
        pltpu.sync_copy(dst_hbm.at[pl.ds(sid * nchunk, nchunk)], didx)
        for b in range(NRING):
            pltpu.async_copy(h_c.at[sidx.at[b]], rows[b], sem)
        plsc.subcore_barrier()

        def outer(g, carry):
            for b in range(NRING):
                jj = g * NRING + b
                rbuf = rows[b]
                pltpu.make_async_copy(h_c.at[sidx.at[jj]], rbuf, sem).wait()
                pltpu.sync_copy(rbuf, acc.at[didx.at[jj]], add=True)

                @pl.when(jj + NRING < nchunk)
                def _():
                    pltpu.async_copy(h_c.at[sidx.at[jj + NRING]], rbuf, sem)

            return carry

        lax.fori_loop(0, nchunk // NRING, outer, 0)
        plsc.subcore_barrier()

        out_c = out_hbm.at[cid]
        for kk in range(rows_per_tile // ZR):
            r = row0 + kk * ZR
            pltpu.sync_copy(acc.at[pl.ds(r, ZR)], out_c.at[pl.ds(r, ZR)])

    return k(h2, src2d, dst2d)


def _indeg_sc(dst2d):
    d = LANES
    ew = EP // (NC * NS)
    nchunk = ew // CHUNK
    rows_per_tile = NP // NS
    mesh = plsc.VectorSubcoreMesh(
        core_axis_name="c", subcore_axis_name="s", num_cores=NC, num_subcores=NS
    )

    @functools.partial(
        pl.kernel,
        out_type=jax.ShapeDtypeStruct((NC, NP, d), jnp.float32),
        mesh=mesh,
        scratch_types=[
            pltpu.VMEM((nchunk, CHUNK), jnp.int32),
            pltpu.VMEM((CHUNK, d), jnp.float32),
            pltpu.VMEM((ZR, d), jnp.float32),
            pltpu.VMEM_SHARED((NP, d), jnp.float32),
        ],
        compiler_params=pltpu.CompilerParams(use_tc_tiling_on_sc=False),
    )
    def k(dst_hbm, out_hbm, didx, ones, zbuf, acc):
        cid = lax.axis_index("c")
        sid = lax.axis_index("s")
        wid = sid * NC + cid
        z16 = jnp.zeros((LANES,), jnp.float32)
        o16 = jnp.ones((LANES,), jnp.float32)

        def zrow(i, carry):
            zbuf[i, pl.ds(0, LANES)] = z16
            ones[i, pl.ds(0, LANES)] = o16
            return carry

        lax.fori_loop(0, ZR, zrow, 0)

        row0 = sid * rows_per_tile
        for kk in range(rows_per_tile // ZR):
            pltpu.sync_copy(zbuf, acc.at[pl.ds(row0 + kk * ZR, ZR)])
        pltpu.sync_copy(dst_hbm.at[pl.ds(wid * nchunk, nchunk)], didx)
        plsc.subcore_barrier()

        def body(j, carry):
            pltpu.sync_copy(ones, acc.at[didx.at[j]], add=True)
            return carry

        lax.fori_loop(0, nchunk, body, 0)
        plsc.subcore_barrier()

        out_c = out_hbm.at[cid]
        for kk in range(rows_per_tile // ZR):
            r = row0 + kk * ZR
            pltpu.sync_copy(acc.at[pl.ds(r, ZR)], out_c.at[pl.ds(r, ZR)])

    return k(dst2d)


_BN = 1000


def _tc_first(x, w, ind):
    n, din = x.shape
    dh = w.shape[1]
    ng = dh // DG

    def body(x_ref, w_ref, ind_ref, dis_ref, hs_ref):
        indeg = ind_ref[0, :, :1] + ind_ref[1, :, :1]
        dis = lax.rsqrt(indeg + 1.0)
        dis_ref[...] = dis
        h = jnp.dot(x_ref[...], w_ref[...], preferred_element_type=jnp.float32)
        hs = h * dis
        for g in range(ng):
            hs_ref[g] = hs[:, g * DG:(g + 1) * DG]

    return pl.pallas_call(
        body,
        grid=(n // _BN,),
        in_specs=[
            pl.BlockSpec((_BN, din), lambda i: (i, 0)),
            pl.BlockSpec((din, dh), lambda i: (0, 0)),
            pl.BlockSpec((NC, _BN, LANES), lambda i: (0, i, 0)),
        ],
        out_specs=[
            pl.BlockSpec((_BN, 1), lambda i: (i, 0)),
            pl.BlockSpec((ng, _BN, DG), lambda i: (0, i, 0)),
        ],
        out_shape=[
            jax.ShapeDtypeStruct((n, 1), jnp.float32),
            jax.ShapeDtypeStruct((ng, n, DG), jnp.float32),
        ],
    )(x, w, ind)


def _tc_mid(s_parts, hs, b, dis, w):
    npart = len(s_parts)
    ng, n, _ = hs.shape
    dn = w.shape[1]
    og = dn // DG

    def body(*refs):
        s_refs = refs[:npart]
        hs_ref, b_ref, dis_ref, w_ref, out_ref = refs[npart:]
        agg = jnp.concatenate(
            [s_refs[g // 2][g % 2] + hs_ref[g] for g in range(ng)], axis=-1
        )
        xv = agg * dis_ref[...] + b_ref[...]
        xv = jnp.maximum(xv, 0.0)
        y = (
            jnp.dot(xv, w_ref[...], preferred_element_type=jnp.float32)
            * dis_ref[...]
        )
        for g in range(og):
            out_ref[g] = y[:, g * DG:(g + 1) * DG]

    return pl.pallas_call(
        body,
        grid=(n // _BN,),
        in_specs=[pl.BlockSpec((NC, _BN, DG), lambda i: (0, i, 0))] * npart
        + [
            pl.BlockSpec((ng, _BN, DG), lambda i: (0, i, 0)),
            pl.BlockSpec((1, ng * DG), lambda i: (0, 0)),
            pl.BlockSpec((_BN, 1), lambda i: (i, 0)),
            pl.BlockSpec((ng * DG, dn), lambda i: (0, 0)),
        ],
        out_specs=pl.BlockSpec((og, _BN, DG), lambda i: (0, i, 0)),
        out_shape=jax.ShapeDtypeStruct((og, n, DG), jnp.float32),
    )(*s_parts, hs, b, dis, w)


def _tc_final(s, hs, b, dis):
    ng, n, _ = hs.shape

    def body(s_ref, hs_ref, b_ref, dis_ref, out_ref):
        agg = jnp.concatenate(
            [s_ref[g] + hs_ref[g] for g in range(ng)], axis=-1
        )
        out_ref[...] = agg * dis_ref[...] + b_ref[...]

    return pl.pallas_call(
        body,
        grid=(n // _BN,),
        in_specs=[
            pl.BlockSpec((NC, _BN, DG), lambda i: (0, i, 0)),
            pl.BlockSpec((ng, _BN, DG), lambda i: (0, i, 0)),
            pl.BlockSpec((1, ng * DG), lambda i: (0, 0)),
            pl.BlockSpec((_BN, 1), lambda i: (i, 0)),
        ],
        out_specs=pl.BlockSpec((_BN, ng * DG), lambda i: (i, 0)),
        out_shape=jax.ShapeDtypeStruct((n, ng * DG), jnp.float32),
    )(s, hs, b, dis)


def kernel(x, edge_index, W1, b1, W2, b2, W3, b3):
    e = edge_index.shape[1]
    pad = EP - e
    src2d = jnp.concatenate(
        [edge_index[0], jnp.zeros((pad,), jnp.int32)]
    ).reshape(EP // CHUNK, CHUNK)
    dst2d = jnp.concatenate(
        [edge_index[1], jnp.full((pad,), NP - 1, jnp.int32)]
    ).reshape(EP // CHUNK, CHUNK)

    ind = _indeg_sc(dst2d)
    dis, hs1 = _tc_first(x, W1, ind)

    s1a = _seg_sum_sc(hs1[0:2], src2d, dst2d)
    s1b = _seg_sum_sc(hs1[2:4], src2d, dst2d)
    hs2 = _tc_mid([s1a, s1b], hs1, b1.reshape(1, -1), dis, W2)

    s2 = _seg_sum_sc(hs2, src2d, dst2d)
    hs3 = _tc_mid([s2], hs2, b2.reshape(1, -1), dis, W3)

    s3 = _seg_sum_sc(hs3, src2d, dst2d)
    return _tc_final(s3, hs3, b3.reshape(1, -1), dis)
